# baseline (speedup 1.0000x reference)
import functools

import jax
import jax.numpy as jnp
from jax import lax
from jax.experimental import pallas as pl
from jax.experimental.pallas import tpu as pltpu
from jax.experimental.pallas import tpu_sc as plsc

B = 16
T = 11
S = 512
H = 1024
BB = 4
L = 16
HC = H // L


CIN = 3 * H + L
COUT = L + H


def _sc_body(combo_hbm, out_hbm, combo_v, out_v):
    i = lax.axis_index("s")

    pltpu.sync_copy(combo_hbm.at[i], combo_v)

    acc = jnp.zeros((L,), jnp.float32)
    for c in range(HC):
        acc = acc + combo_v[pl.ds(c * L, L)] * combo_v[pl.ds(2 * H + c * L, L)]
    l = jnp.sum(acc)

    lane = lax.broadcasted_iota(jnp.int32, (L,), 0).astype(jnp.float32)
    sel = (lane == i.astype(jnp.float32)).astype(jnp.float32)
    sm_i = jnp.sum(combo_v[pl.ds(3 * H, L)] * sel)

    logits = jnp.where(lane == float(T - 1), l, 0.0)
    maskv = ((lane >= float(T) - sm_i) & (lane <= float(T - 1)))
    e = jnp.exp(logits) * maskv.astype(jnp.float32)
    probs = e / jnp.sum(e)
    out_v[pl.ds(0, L)] = probs

    p = jnp.sum(probs * jnp.where(lane == float(T - 1), 1.0, 0.0))
    for c in range(HC):
        out_v[pl.ds(L + c * L, L)] = combo_v[pl.ds(H + c * L, L)] * p

    pltpu.sync_copy(out_v, out_hbm.at[i])


def _sc_probs_mtl(combo):
    mesh = plsc.VectorSubcoreMesh(
        core_axis_name="c", subcore_axis_name="s", num_cores=1)
    fn = functools.partial(
        pl.kernel, _sc_body, mesh=mesh,
        compiler_params=pltpu.CompilerParams(
            needs_layout_passes=False, skip_device_barrier=True),
        out_type=jax.ShapeDtypeStruct((B, COUT), jnp.float32),
        scratch_types=[
            pltpu.VMEM((CIN,), jnp.float32),
            pltpu.VMEM((COUT,), jnp.float32),
        ],
    )()
    return fn(combo)


def _tc_step(bert_ref, hai_ref, w_ref, smf_ref, bert_out_ref):
    l = jnp.sum(hai_ref[:, 0, :] * w_ref[...], axis=1, keepdims=True)
    t = lax.broadcasted_iota(jnp.int32, (1, T), 1).astype(jnp.float32)
    logits = jnp.where(t == float(T - 1), l, 0.0)
    mask = (t >= (float(T) - smf_ref[:, 0, :])).astype(jnp.float32)
    e = jnp.exp(logits) * mask
    p = (e[:, T - 1] / jnp.sum(e, axis=1)).reshape(BB, 1, 1)
    bert_out_ref[...] = bert_ref[...] * p


def kernel(bert_representation, mtl_input, slice_mask, slice_num,
           history_attention_input, W):
    del slice_num
    smf = slice_mask.astype(jnp.float32)
    hai3 = history_attention_input.reshape(B, 1, H)

    combo = jnp.concatenate(
        [history_attention_input, mtl_input,
         jnp.broadcast_to(W, (B, H)),
         jnp.broadcast_to(smf, (B, L))], axis=1)
    sc_out = _sc_probs_mtl(combo)
    probs16 = sc_out[:, :L]
    new_mtl = sc_out[:, L:]

    bert_out = pl.pallas_call(
        _tc_step,
        grid=(B // BB,),
        in_specs=[
            pl.BlockSpec((BB, S, H), lambda i: (i, 0, 0)),
            pl.BlockSpec((BB, 1, H), lambda i: (i, 0, 0)),
            pl.BlockSpec((1, H), lambda i: (0, 0)),
            pl.BlockSpec((BB, 1, 1), lambda i: (i, 0, 0)),
        ],
        out_specs=pl.BlockSpec((BB, S, H), lambda i: (i, 0, 0)),
        out_shape=jax.ShapeDtypeStruct((B, S, H), jnp.float32),
    )(bert_representation, hai3, W, smf.reshape(B, 1, 1))

    return (bert_out, new_mtl, probs16[:, :T])


# speedup vs baseline: 1.0685x; 1.0685x over previous
import functools

import jax
import jax.numpy as jnp
from jax import lax
from jax.experimental import pallas as pl
from jax.experimental.pallas import tpu as pltpu
from jax.experimental.pallas import tpu_sc as plsc

B = 16
T = 11
S = 512
H = 1024
BB = 4
L = 16
HC = H // L


def _sc_body(hai_hbm, mtl_hbm, w_hbm, smf_hbm,
             probs_hbm, mtl_out_hbm,
             hai_v, w_v, mtl_v, out_v, vec_v, smf_v, sem):
    i = lax.axis_index("s")

    copies = [
        pltpu.async_copy(hai_hbm.at[i], hai_v, sem),
        pltpu.async_copy(w_hbm, w_v, sem),
        pltpu.async_copy(mtl_hbm.at[i], mtl_v, sem),
        pltpu.async_copy(smf_hbm, smf_v, sem),
    ]
    for c in copies:
        c.wait()

    acc = jnp.zeros((L,), jnp.float32)
    for c in range(HC):
        acc = acc + hai_v[pl.ds(c * L, L)] * w_v[pl.ds(c * L, L)]
    l = jnp.sum(acc)

    lane = lax.broadcasted_iota(jnp.int32, (L,), 0).astype(jnp.float32)
    sel = (lane == i.astype(jnp.float32)).astype(jnp.float32)
    sm_i = jnp.sum(smf_v[...] * sel)

    logits = jnp.where(lane == float(T - 1), l, 0.0)
    maskv = ((lane >= float(T) - sm_i) & (lane <= float(T - 1)))
    e = jnp.exp(logits) * maskv.astype(jnp.float32)
    probs = e / jnp.sum(e)
    vec_v[...] = probs

    p = jnp.sum(probs * jnp.where(lane == float(T - 1), 1.0, 0.0))
    for c in range(HC):
        out_v[pl.ds(c * L, L)] = mtl_v[pl.ds(c * L, L)] * p

    outs = [
        pltpu.async_copy(vec_v, probs_hbm.at[i], sem),
        pltpu.async_copy(out_v, mtl_out_hbm.at[i], sem),
    ]
    for c in outs:
        c.wait()


def _sc_probs_mtl(hai, mtl, w_flat, smf):
    mesh = plsc.VectorSubcoreMesh(
        core_axis_name="c", subcore_axis_name="s", num_cores=1)
    fn = functools.partial(
        pl.kernel, _sc_body, mesh=mesh,
        compiler_params=pltpu.CompilerParams(
            needs_layout_passes=False, skip_device_barrier=True),
        out_type=(
            jax.ShapeDtypeStruct((B, L), jnp.float32),
            jax.ShapeDtypeStruct((B, H), jnp.float32),
        ),
        scratch_types=[
            pltpu.VMEM((H,), jnp.float32),
            pltpu.VMEM((H,), jnp.float32),
            pltpu.VMEM((H,), jnp.float32),
            pltpu.VMEM((H,), jnp.float32),
            pltpu.VMEM((L,), jnp.float32),
            pltpu.VMEM((L,), jnp.float32),
            pltpu.SemaphoreType.DMA,
        ],
    )()
    return fn(hai, mtl, w_flat, smf)


def _tc_step(bert_ref, hai_ref, w_ref, smf_ref, bert_out_ref):
    l = jnp.sum(hai_ref[:, 0, :] * w_ref[...], axis=1, keepdims=True)
    t = lax.broadcasted_iota(jnp.int32, (1, T), 1).astype(jnp.float32)
    logits = jnp.where(t == float(T - 1), l, 0.0)
    mask = (t >= (float(T) - smf_ref[:, 0, :])).astype(jnp.float32)
    e = jnp.exp(logits) * mask
    p = (e[:, T - 1] / jnp.sum(e, axis=1)).reshape(BB, 1, 1)
    bert_out_ref[...] = bert_ref[...] * p


def kernel(bert_representation, mtl_input, slice_mask, slice_num,
           history_attention_input, W):
    del slice_num
    smf = slice_mask.astype(jnp.float32)
    hai3 = history_attention_input.reshape(B, 1, H)

    probs16, new_mtl = _sc_probs_mtl(
        history_attention_input, mtl_input, W.reshape(H), smf)

    bert_out = pl.pallas_call(
        _tc_step,
        grid=(B // BB,),
        in_specs=[
            pl.BlockSpec((BB, S, H), lambda i: (i, 0, 0)),
            pl.BlockSpec((BB, 1, H), lambda i: (i, 0, 0)),
            pl.BlockSpec((1, H), lambda i: (0, 0)),
            pl.BlockSpec((BB, 1, 1), lambda i: (i, 0, 0)),
        ],
        out_specs=pl.BlockSpec((BB, S, H), lambda i: (i, 0, 0)),
        out_shape=jax.ShapeDtypeStruct((B, S, H), jnp.float32),
    )(bert_representation, hai3, W, smf.reshape(B, 1, 1))

    return (bert_out, new_mtl, probs16[:, :T])


# speedup vs baseline: 1.0758x; 1.0069x over previous
import functools

import jax
import jax.numpy as jnp
from jax import lax
from jax.experimental import pallas as pl
from jax.experimental.pallas import tpu as pltpu
from jax.experimental.pallas import tpu_sc as plsc

B = 16
T = 11
S = 512
H = 1024
BB = 4
L = 16
HC = H // L


def _sc_body(hai_hbm, mtl_hbm, w_hbm, smf_hbm,
             probs_hbm, mtl_out_hbm,
             hai_v, w_v, mtl_v, out_v, vec_v, smf_v, sem):
    i = lax.axis_index("s")

    copies = [
        pltpu.async_copy(hai_hbm.at[i], hai_v, sem),
        pltpu.async_copy(w_hbm, w_v, sem),
        pltpu.async_copy(mtl_hbm.at[i], mtl_v, sem),
        pltpu.async_copy(smf_hbm, smf_v, sem),
    ]
    for c in copies:
        c.wait()

    acc = jnp.zeros((L,), jnp.float32)
    for c in range(HC):
        acc = acc + hai_v[pl.ds(c * L, L)] * w_v[pl.ds(c * L, L)]
    l = jnp.sum(acc)

    lane = lax.broadcasted_iota(jnp.int32, (L,), 0).astype(jnp.float32)
    sel = (lane == i.astype(jnp.float32)).astype(jnp.float32)
    sm_i = jnp.sum(smf_v[...].astype(jnp.float32) * sel)

    logits = jnp.where(lane == float(T - 1), l, 0.0)
    maskv = ((lane >= float(T) - sm_i) & (lane <= float(T - 1)))
    e = jnp.exp(logits) * maskv.astype(jnp.float32)
    probs = e / jnp.sum(e)
    vec_v[...] = probs

    p = jnp.sum(probs * jnp.where(lane == float(T - 1), 1.0, 0.0))
    for c in range(HC):
        out_v[pl.ds(c * L, L)] = mtl_v[pl.ds(c * L, L)] * p

    outs = [
        pltpu.async_copy(vec_v, probs_hbm.at[i], sem),
        pltpu.async_copy(out_v, mtl_out_hbm.at[i], sem),
    ]
    for c in outs:
        c.wait()


def _sc_probs_mtl(hai, mtl, w_flat, smi):
    mesh = plsc.VectorSubcoreMesh(
        core_axis_name="c", subcore_axis_name="s", num_cores=1)
    fn = functools.partial(
        pl.kernel, _sc_body, mesh=mesh,
        compiler_params=pltpu.CompilerParams(
            needs_layout_passes=False, skip_device_barrier=True),
        out_type=(
            jax.ShapeDtypeStruct((B, L), jnp.float32),
            jax.ShapeDtypeStruct((B, H), jnp.float32),
        ),
        scratch_types=[
            pltpu.VMEM((H,), jnp.float32),
            pltpu.VMEM((H,), jnp.float32),
            pltpu.VMEM((H,), jnp.float32),
            pltpu.VMEM((H,), jnp.float32),
            pltpu.VMEM((L,), jnp.float32),
            pltpu.VMEM((L,), jnp.int32),
            pltpu.SemaphoreType.DMA,
        ],
    )()
    return fn(hai, mtl, w_flat, smi)


def _tc_step(bert_ref, hai_ref, w_ref, smi_ref, bert_out_ref):
    l = jnp.sum(hai_ref[:, 0, :] * w_ref[...], axis=1, keepdims=True)
    t = lax.broadcasted_iota(jnp.int32, (1, T), 1).astype(jnp.float32)
    logits = jnp.where(t == float(T - 1), l, 0.0)
    smf = smi_ref[:, 0, :].astype(jnp.float32)
    mask = (t >= (float(T) - smf)).astype(jnp.float32)
    e = jnp.exp(logits) * mask
    p = (e[:, T - 1] / jnp.sum(e, axis=1)).reshape(BB, 1, 1)
    bert_out_ref[...] = bert_ref[...] * p


def kernel(bert_representation, mtl_input, slice_mask, slice_num,
           history_attention_input, W):
    del slice_num
    smi = slice_mask.astype(jnp.int32)
    hai3 = history_attention_input.reshape(B, 1, H)

    probs16, new_mtl = _sc_probs_mtl(
        history_attention_input, mtl_input, W.reshape(H), smi)

    bert_out = pl.pallas_call(
        _tc_step,
        grid=(B // BB,),
        in_specs=[
            pl.BlockSpec((BB, S, H), lambda i: (i, 0, 0)),
            pl.BlockSpec((BB, 1, H), lambda i: (i, 0, 0)),
            pl.BlockSpec((1, H), lambda i: (0, 0)),
            pl.BlockSpec((BB, 1, 1), lambda i: (i, 0, 0)),
        ],
        out_specs=pl.BlockSpec((BB, S, H), lambda i: (i, 0, 0)),
        out_shape=jax.ShapeDtypeStruct((B, S, H), jnp.float32),
    )(bert_representation, hai3, W, smi.reshape(B, 1, 1))

    return (bert_out, new_mtl, probs16[:, :T])
